# baseline (speedup 1.0000x reference)
import dataclasses

import jax
import jax.numpy as jnp
from jax import lax
from jax.experimental import pallas as pl
from jax.experimental.pallas import tpu as pltpu
from jax.experimental.pallas import tpu_sc as plsc

N = 10000
E = 320000
D = 128
LAM = 0.5

NC = 2
NS = 16
NW = NC * NS
CHUNK = 128
E_PAD = 327680
EDGES_PER_TILE = E_PAD // NW
CHUNKS_PER_TILE = EDGES_PER_TILE // CHUNK
NPAD = N + 8
ZSTEP = 624
ZSPAN = 640

ROW_BLOCK = 1000


def _mlp_body(h_ref, w_ref, b_ref, z_ref):
    z = lax.dot_general(
        h_ref[...], w_ref[...], (((1,), (0,)), ((), ())),
        preferred_element_type=jnp.float32,
        precision=lax.Precision.HIGHEST,
    )
    z = jnp.maximum(z + b_ref[...], 0.0)
    nrm = jnp.sqrt(jnp.sum(z * z, axis=1, keepdims=True))
    z_ref[...] = z / jnp.maximum(nrm, 1e-12)


def _mlp(h, W, b2d):
    return pl.pallas_call(
        _mlp_body,
        grid=(N // ROW_BLOCK,),
        in_specs=[
            pl.BlockSpec((ROW_BLOCK, D), lambda i: (i, 0)),
            pl.BlockSpec((D, D), lambda i: (0, 0)),
            pl.BlockSpec((1, D), lambda i: (0, 0)),
        ],
        out_specs=pl.BlockSpec((ROW_BLOCK, D), lambda i: (i, 0)),
        out_shape=jax.ShapeDtypeStruct((N, D), jnp.float32),
    )(h, W, b2d)


CHG = 1
GW = CHG * CHUNK
NG = CHUNKS_PER_TILE // CHG


def _make_hop(with_deg):
    mesh = plsc.VectorSubcoreMesh(core_axis_name="c", subcore_axis_name="s")

    out_type = [jax.ShapeDtypeStruct((NC, N, D), jnp.float32)]
    scratch = [
        pltpu.VMEM((2, CHUNK), jnp.int32),
        pltpu.VMEM((CHUNK, D), jnp.float32),
        pltpu.VMEM_SHARED((NPAD, D), jnp.float32),
    ]
    if with_deg:
        out_type.append(jax.ShapeDtypeStruct((NW, 8, NPAD), jnp.float32))
        scratch.append(pltpu.VMEM((NPAD,), jnp.float32))

    def inner(table, sd3, zrows, out, degout, idxsd, rows, acc, degt):
        c = lax.axis_index("c")
        s = lax.axis_index("s")
        w = c * NS + s
        row0 = pl.multiple_of(s * ZSTEP, 8)
        gbase = w * CHUNKS_PER_TILE
        pltpu.sync_copy(zrows, acc.at[pl.ds(row0, ZSPAN)])
        if with_deg:
            @pl.loop(0, NPAD // 16)
            def _(i):
                degt[pl.ds(pl.multiple_of(i * 16, 16), 16)] = jnp.zeros(
                    (16,), jnp.float32)
        plsc.subcore_barrier()

        @pl.loop(0, CHUNKS_PER_TILE)
        def _(i):
            pltpu.sync_copy(sd3.at[gbase + i], idxsd)
            pltpu.sync_copy(table.at[idxsd.at[0]], rows)
            pltpu.sync_copy(rows, acc.at[idxsd.at[1]], add=True)
            if with_deg:
                for t in range(CHUNK // 16):
                    iv = idxsd[1, pl.ds(t * 16, 16)]
                    plsc.addupdate_scatter(degt, [iv],
                                           jnp.ones((16,), jnp.float32))

        plsc.subcore_barrier()
        pltpu.sync_copy(acc.at[pl.ds(row0, ZSPAN)],
                        out.at[c, pl.ds(row0, ZSPAN)])
        if with_deg:
            pltpu.sync_copy(degt, degout.at[w, 0])

    if with_deg:
        def body(table, sd3, zrows, out, degout, idxsd, rows, acc, degt):
            inner(table, sd3, zrows, out, degout, idxsd, rows, acc, degt)
    else:
        def body(table, sd3, zrows, out, idxsd, rows, acc):
            inner(table, sd3, zrows, out, None, idxsd, rows, acc, None)

    cp = pltpu.CompilerParams()
    if "needs_layout_passes" in pltpu.CompilerParams.__dataclass_fields__:
        cp = dataclasses.replace(cp, needs_layout_passes=False)
    return pl.kernel(body, out_type=out_type, mesh=mesh,
                     scratch_types=scratch, compiler_params=cp)


_hop_deg = _make_hop(True)
_hop = _make_hop(False)


def _c1_body(p_ref, pd_ref, out_ref):
    s = p_ref[0] + p_ref[1]
    deg = jnp.sum(pd_ref[:, 0, :], axis=0)[:N]
    out_ref[...] = s / jnp.maximum(deg, 1.0)[:, None]


def _combine1(p, pdeg):
    return pl.pallas_call(
        _c1_body,
        grid=(1,),
        in_specs=[
            pl.BlockSpec((NC, N, D), lambda i: (0, 0, 0)),
            pl.BlockSpec((NW, 8, NPAD), lambda i: (0, 0, 0)),
        ],
        out_specs=pl.BlockSpec((N, D), lambda i: (0, 0)),
        out_shape=jax.ShapeDtypeStruct((N, D), jnp.float32),
    )(p, pdeg)


def _c2_body(n1_ref, p_ref, pd_ref, out_ref):
    s = p_ref[0] + p_ref[1]
    deg = jnp.sum(pd_ref[:, 0, :], axis=0)[:N]
    neigh2 = s / jnp.maximum(deg, 1.0)[:, None]
    out_ref[...] = LAM * n1_ref[...] + (1.0 - LAM) * neigh2


def _combine2(n1, p, pdeg):
    return pl.pallas_call(
        _c2_body,
        grid=(1,),
        in_specs=[
            pl.BlockSpec((N, D), lambda i: (0, 0)),
            pl.BlockSpec((NC, N, D), lambda i: (0, 0, 0)),
            pl.BlockSpec((NW, 8, NPAD), lambda i: (0, 0, 0)),
        ],
        out_specs=pl.BlockSpec((N, D), lambda i: (0, 0)),
        out_shape=jax.ShapeDtypeStruct((N, D), jnp.float32),
    )(n1, p, pdeg)


def kernel(h, edge_index, W, b):
    z = _mlp(h, W, b.reshape(1, D))
    pad = E_PAD - E
    srcp = jnp.concatenate([edge_index[0],
                            jnp.zeros((pad,), jnp.int32)])
    dstp = jnp.concatenate([edge_index[1],
                            jnp.full((pad,), N, jnp.int32)])
    sd3 = jnp.stack([srcp.reshape(E_PAD // CHUNK, CHUNK),
                     dstp.reshape(E_PAD // CHUNK, CHUNK)], axis=1)
    zrows = jnp.zeros((ZSPAN, D), jnp.float32)
    p1, pdeg = _hop_deg(z, sd3, zrows)
    neigh1 = _combine1(p1, pdeg)
    (p2,) = _hop(neigh1, sd3, zrows)
    result = _combine2(neigh1, p2, pdeg)
    return (z, result)


# speedup vs baseline: 1.2417x; 1.2417x over previous
import dataclasses

import jax
import jax.numpy as jnp
from jax import lax
from jax.experimental import pallas as pl
from jax.experimental.pallas import tpu as pltpu
from jax.experimental.pallas import tpu_sc as plsc

N = 10000
E = 320000
D = 128
LAM = 0.5

NC = 2
NS = 16
NW = NC * NS
CHUNK = 128
E_PAD = 327680
EDGES_PER_TILE = E_PAD // NW
CHUNKS_PER_TILE = EDGES_PER_TILE // CHUNK
NPAD = N + 8
ZSTEP = 624
ZSPAN = 640

ROW_BLOCK = 1000


def _mlp_body(h_ref, w_ref, b_ref, z_ref):
    z = lax.dot_general(
        h_ref[...], w_ref[...], (((1,), (0,)), ((), ())),
        preferred_element_type=jnp.float32,
        precision=lax.Precision.HIGHEST,
    )
    z = jnp.maximum(z + b_ref[...], 0.0)
    nrm = jnp.sqrt(jnp.sum(z * z, axis=1, keepdims=True))
    z_ref[...] = z / jnp.maximum(nrm, 1e-12)


def _mlp(h, W, b2d):
    return pl.pallas_call(
        _mlp_body,
        grid=(N // ROW_BLOCK,),
        in_specs=[
            pl.BlockSpec((ROW_BLOCK, D), lambda i: (i, 0)),
            pl.BlockSpec((D, D), lambda i: (0, 0)),
            pl.BlockSpec((1, D), lambda i: (0, 0)),
        ],
        out_specs=pl.BlockSpec((ROW_BLOCK, D), lambda i: (i, 0)),
        out_shape=jax.ShapeDtypeStruct((N, D), jnp.float32),
    )(h, W, b2d)


CHG = 1
GW = CHG * CHUNK
NG = CHUNKS_PER_TILE // CHG


def _make_hop(with_deg):
    mesh = plsc.VectorSubcoreMesh(core_axis_name="c", subcore_axis_name="s")

    out_type = [jax.ShapeDtypeStruct((NC, N, D), jnp.float32)]
    scratch = [
        pltpu.VMEM((2, CHUNK), jnp.int32),
        pltpu.VMEM((CHUNK, D), jnp.float32),
        pltpu.VMEM_SHARED((NPAD, D), jnp.float32),
    ]
    if with_deg:
        out_type.append(jax.ShapeDtypeStruct((NW, 8, NPAD), jnp.float32))
        scratch.append(pltpu.VMEM((NPAD,), jnp.float32))

    def inner(table, sd3, zrows, out, degout, idxsd, rows, acc, degt):
        c = lax.axis_index("c")
        s = lax.axis_index("s")
        w = c * NS + s
        row0 = pl.multiple_of(s * ZSTEP, 8)
        gbase = w * CHUNKS_PER_TILE
        pltpu.sync_copy(zrows, acc.at[pl.ds(row0, ZSPAN)])
        if with_deg:
            @pl.loop(0, NPAD // 16)
            def _(i):
                degt[pl.ds(pl.multiple_of(i * 16, 16), 16)] = jnp.zeros(
                    (16,), jnp.float32)
        plsc.subcore_barrier()

        @pl.loop(0, CHUNKS_PER_TILE)
        def _(i):
            pltpu.sync_copy(sd3.at[gbase + i], idxsd)
            pltpu.sync_copy(table.at[idxsd.at[0]], rows)
            pltpu.sync_copy(rows, acc.at[idxsd.at[1]], add=True)
            if with_deg:
                for t in range(CHUNK // 16):
                    iv = idxsd[1, pl.ds(t * 16, 16)]
                    plsc.addupdate_scatter(degt, [iv],
                                           jnp.ones((16,), jnp.float32))

        plsc.subcore_barrier()
        pltpu.sync_copy(acc.at[pl.ds(row0, ZSPAN)],
                        out.at[c, pl.ds(row0, ZSPAN)])
        if with_deg:
            pltpu.sync_copy(degt, degout.at[w, 0])

    if with_deg:
        def body(table, sd3, zrows, out, degout, idxsd, rows, acc, degt):
            inner(table, sd3, zrows, out, degout, idxsd, rows, acc, degt)
    else:
        def body(table, sd3, zrows, out, idxsd, rows, acc):
            inner(table, sd3, zrows, out, None, idxsd, rows, acc, None)

    cp = pltpu.CompilerParams()
    if "needs_layout_passes" in pltpu.CompilerParams.__dataclass_fields__:
        cp = dataclasses.replace(cp, needs_layout_passes=False)
    return pl.kernel(body, out_type=out_type, mesh=mesh,
                     scratch_types=scratch, compiler_params=cp)


_hop_deg = _make_hop(True)
_hop = _make_hop(False)


def _c1_body(p_ref, pd_ref, out_ref):
    s = p_ref[0] + p_ref[1]
    deg = jnp.sum(pd_ref[:, 0, :], axis=0)[:N]
    out_ref[...] = s / jnp.maximum(deg, 1.0)[:, None]


def _combine1(p, pdeg):
    return pl.pallas_call(
        _c1_body,
        grid=(1,),
        in_specs=[
            pl.BlockSpec((NC, N, D), lambda i: (0, 0, 0)),
            pl.BlockSpec((NW, 8, NPAD), lambda i: (0, 0, 0)),
        ],
        out_specs=pl.BlockSpec((N, D), lambda i: (0, 0)),
        out_shape=jax.ShapeDtypeStruct((N, D), jnp.float32),
    )(p, pdeg)


def _c2_body(n1_ref, p_ref, pd_ref, out_ref):
    s = p_ref[0] + p_ref[1]
    deg = jnp.sum(pd_ref[:, 0, :], axis=0)[:N]
    neigh2 = s / jnp.maximum(deg, 1.0)[:, None]
    out_ref[...] = LAM * n1_ref[...] + (1.0 - LAM) * neigh2


def _combine2(n1, p, pdeg):
    return pl.pallas_call(
        _c2_body,
        grid=(1,),
        in_specs=[
            pl.BlockSpec((N, D), lambda i: (0, 0)),
            pl.BlockSpec((NC, N, D), lambda i: (0, 0, 0)),
            pl.BlockSpec((NW, 8, NPAD), lambda i: (0, 0, 0)),
        ],
        out_specs=pl.BlockSpec((N, D), lambda i: (0, 0)),
        out_shape=jax.ShapeDtypeStruct((N, D), jnp.float32),
    )(n1, p, pdeg)


def kernel(h, edge_index, W, b):
    z = _mlp(h, W, b.reshape(1, D))
    pad_per_tile = EDGES_PER_TILE - E // NW
    pad_src = jnp.zeros((NW, pad_per_tile), jnp.int32)
    pad_dst = jnp.broadcast_to(
        N + (jnp.arange(pad_per_tile, dtype=jnp.int32) % 8),
        (NW, pad_per_tile))
    srcp = jnp.concatenate(
        [edge_index[0].reshape(NW, E // NW), pad_src], axis=1)
    dstp = jnp.concatenate(
        [edge_index[1].reshape(NW, E // NW), pad_dst], axis=1)
    sd3 = jnp.stack([srcp.reshape(E_PAD // CHUNK, CHUNK),
                     dstp.reshape(E_PAD // CHUNK, CHUNK)], axis=1)
    zrows = jnp.zeros((ZSPAN, D), jnp.float32)
    p1, pdeg = _hop_deg(z, sd3, zrows)
    neigh1 = _combine1(p1, pdeg)
    (p2,) = _hop(neigh1, sd3, zrows)
    result = _combine2(neigh1, p2, pdeg)
    return (z, result)


# speedup vs baseline: 2.1659x; 1.7443x over previous
import dataclasses

import jax
import jax.numpy as jnp
from jax import lax
from jax.experimental import pallas as pl
from jax.experimental.pallas import tpu as pltpu
from jax.experimental.pallas import tpu_sc as plsc

N = 10000
E = 320000
D = 128
LAM = 0.5

NC = 2
NS = 16
NW = NC * NS
CHUNK = 80
E_PAD = E
EDGES_PER_TILE = E_PAD // NW
CHUNKS_PER_TILE = EDGES_PER_TILE // CHUNK
NPAD = N
ZSTEP = 624
ZSPAN = 640

ROW_BLOCK = 1000


def _mlp_body(h_ref, w_ref, b_ref, z_ref):
    z = lax.dot_general(
        h_ref[...], w_ref[...], (((1,), (0,)), ((), ())),
        preferred_element_type=jnp.float32,
        precision=lax.Precision.HIGHEST,
    )
    z = jnp.maximum(z + b_ref[...], 0.0)
    nrm = jnp.sqrt(jnp.sum(z * z, axis=1, keepdims=True))
    z_ref[...] = z / jnp.maximum(nrm, 1e-12)


def _mlp(h, W, b2d):
    return pl.pallas_call(
        _mlp_body,
        grid=(N // ROW_BLOCK,),
        in_specs=[
            pl.BlockSpec((ROW_BLOCK, D), lambda i: (i, 0)),
            pl.BlockSpec((D, D), lambda i: (0, 0)),
            pl.BlockSpec((1, D), lambda i: (0, 0)),
        ],
        out_specs=pl.BlockSpec((ROW_BLOCK, D), lambda i: (i, 0)),
        out_shape=jax.ShapeDtypeStruct((N, D), jnp.float32),
    )(h, W, b2d)


CHG = 1
GW = CHG * CHUNK
NG = CHUNKS_PER_TILE // CHG


def _make_hop(with_deg):
    mesh = plsc.VectorSubcoreMesh(core_axis_name="c", subcore_axis_name="s")

    out_type = [jax.ShapeDtypeStruct((NC, N, D), jnp.float32)]
    scratch = [
        pltpu.VMEM((2, CHUNK), jnp.int32),
        pltpu.VMEM((CHUNK, D), jnp.float32),
        pltpu.VMEM_SHARED((NPAD, D), jnp.float32),
    ]
    if with_deg:
        out_type.append(jax.ShapeDtypeStruct((NW, 8, NPAD), jnp.float32))
        scratch.append(pltpu.VMEM((NPAD,), jnp.float32))

    def inner(table, sd3, zrows, out, degout, idxsd, rows, acc, degt):
        c = lax.axis_index("c")
        s = lax.axis_index("s")
        w = c * NS + s
        row0 = pl.multiple_of(s * ZSTEP, 8)
        gbase = w * CHUNKS_PER_TILE
        pltpu.sync_copy(zrows, acc.at[pl.ds(row0, ZSPAN)])
        if with_deg:
            @pl.loop(0, NPAD // 16)
            def _(i):
                degt[pl.ds(pl.multiple_of(i * 16, 16), 16)] = jnp.zeros(
                    (16,), jnp.float32)
        plsc.subcore_barrier()

        @pl.loop(0, CHUNKS_PER_TILE)
        def _(i):
            pltpu.sync_copy(sd3.at[gbase + i], idxsd)
            pltpu.sync_copy(table.at[idxsd.at[0]], rows)
            pltpu.sync_copy(rows, acc.at[idxsd.at[1]], add=True)
            if with_deg:
                for t in range(CHUNK // 16):
                    iv = idxsd[1, pl.ds(t * 16, 16)]
                    plsc.addupdate_scatter(degt, [iv],
                                           jnp.ones((16,), jnp.float32))

        plsc.subcore_barrier()
        pltpu.sync_copy(acc.at[pl.ds(row0, ZSPAN)],
                        out.at[c, pl.ds(row0, ZSPAN)])
        if with_deg:
            pltpu.sync_copy(degt, degout.at[w, 0])

    if with_deg:
        def body(table, sd3, zrows, out, degout, idxsd, rows, acc, degt):
            inner(table, sd3, zrows, out, degout, idxsd, rows, acc, degt)
    else:
        def body(table, sd3, zrows, out, idxsd, rows, acc):
            inner(table, sd3, zrows, out, None, idxsd, rows, acc, None)

    cp = pltpu.CompilerParams()
    if "needs_layout_passes" in pltpu.CompilerParams.__dataclass_fields__:
        cp = dataclasses.replace(cp, needs_layout_passes=False)
    return pl.kernel(body, out_type=out_type, mesh=mesh,
                     scratch_types=scratch, compiler_params=cp)


_hop_deg = _make_hop(True)
_hop = _make_hop(False)


def _c1_body(p_ref, pd_ref, out_ref):
    s = p_ref[0] + p_ref[1]
    deg = jnp.sum(pd_ref[:, 0, :], axis=0)[:N]
    out_ref[...] = s / jnp.maximum(deg, 1.0)[:, None]


def _combine1(p, pdeg):
    return pl.pallas_call(
        _c1_body,
        grid=(1,),
        in_specs=[
            pl.BlockSpec((NC, N, D), lambda i: (0, 0, 0)),
            pl.BlockSpec((NW, 8, NPAD), lambda i: (0, 0, 0)),
        ],
        out_specs=pl.BlockSpec((N, D), lambda i: (0, 0)),
        out_shape=jax.ShapeDtypeStruct((N, D), jnp.float32),
    )(p, pdeg)


def _c2_body(n1_ref, p_ref, pd_ref, out_ref):
    s = p_ref[0] + p_ref[1]
    deg = jnp.sum(pd_ref[:, 0, :], axis=0)[:N]
    neigh2 = s / jnp.maximum(deg, 1.0)[:, None]
    out_ref[...] = LAM * n1_ref[...] + (1.0 - LAM) * neigh2


def _combine2(n1, p, pdeg):
    return pl.pallas_call(
        _c2_body,
        grid=(1,),
        in_specs=[
            pl.BlockSpec((N, D), lambda i: (0, 0)),
            pl.BlockSpec((NC, N, D), lambda i: (0, 0, 0)),
            pl.BlockSpec((NW, 8, NPAD), lambda i: (0, 0, 0)),
        ],
        out_specs=pl.BlockSpec((N, D), lambda i: (0, 0)),
        out_shape=jax.ShapeDtypeStruct((N, D), jnp.float32),
    )(n1, p, pdeg)


def kernel(h, edge_index, W, b):
    z = _mlp(h, W, b.reshape(1, D))
    sd3 = jnp.stack([edge_index[0].reshape(E_PAD // CHUNK, CHUNK),
                     edge_index[1].reshape(E_PAD // CHUNK, CHUNK)], axis=1)
    zrows = jnp.zeros((ZSPAN, D), jnp.float32)
    p1, pdeg = _hop_deg(z, sd3, zrows)
    neigh1 = _combine1(p1, pdeg)
    (p2,) = _hop(neigh1, sd3, zrows)
    result = _combine2(neigh1, p2, pdeg)
    return (z, result)
